# Initial kernel scaffold; baseline (speedup 1.0000x reference)
import jax
import jax.numpy as jnp
from jax.experimental import pallas as pl


def kernel(xyz1, xyz2, points1, points2, W1, b1, W2, b2):
    raise NotImplementedError("write your pallas kernel here")


# speedup vs baseline: 20.5284x; 20.5284x over previous
import jax
import jax.numpy as jnp
from jax.experimental import pallas as pl

_NB = 512


def _fp_body(x1_ref, x2t_ref, p1_ref, p2_ref, w1a_ref, w1b_ref, b1_ref,
             w2_ref, b2_ref, out_ref):
    nb = x1_ref.shape[1]
    s = x2t_ref.shape[2]

    x1 = x1_ref[0]
    a0 = x1[:, 0:1]
    a1 = x1[:, 1:2]
    a2 = x1[:, 2:3]
    c0 = x2t_ref[0, 0:1, :]
    c1 = x2t_ref[0, 1:2, :]
    c2 = x2t_ref[0, 2:3, :]

    def _r(v):
        return v.astype(jnp.bfloat16).astype(jnp.float32)

    dot = _r(a0) * _r(c0) + _r(a1) * _r(c1) + _r(a2) * _r(c2)
    ss1 = a0 * a0 + a1 * a1 + a2 * a2
    ss2 = c0 * c0 + c1 * c1 + c2 * c2
    d = (-2.0 * dot + ss1) + ss2

    iota = jax.lax.broadcasted_iota(jnp.int32, (nb, s), 1)
    inf = jnp.float32(jnp.inf)

    m1 = jnp.min(d, axis=1, keepdims=True)
    i1 = jnp.min(jnp.where(d == m1, iota, s), axis=1, keepdims=True)
    dm = jnp.where(iota == i1, inf, d)
    m2 = jnp.min(dm, axis=1, keepdims=True)
    i2 = jnp.min(jnp.where(dm == m2, iota, s), axis=1, keepdims=True)
    dm = jnp.where(iota == i2, inf, dm)
    m3 = jnp.min(dm, axis=1, keepdims=True)
    i3 = jnp.min(jnp.where(dm == m3, iota, s), axis=1, keepdims=True)

    r1 = 1.0 / (m1 + 1e-8)
    r2 = 1.0 / (m2 + 1e-8)
    r3 = 1.0 / (m3 + 1e-8)
    rn = r1 + r2 + r3
    w1 = r1 / rn
    w2 = r2 / rn
    w3 = r3 / rn

    zero = jnp.float32(0.0)
    wmat = (jnp.where(iota == i1, w1, zero)
            + jnp.where(iota == i2, w2, zero)
            + jnp.where(iota == i3, w3, zero))

    interp = jnp.dot(wmat, p2_ref[0], preferred_element_type=jnp.float32,
                     precision=jax.lax.Precision.HIGHEST)

    def _bf(v):
        return v.astype(jnp.bfloat16)

    h = jnp.dot(_bf(p1_ref[0]), _bf(w1a_ref[...]),
                preferred_element_type=jnp.float32)
    h = h + jnp.dot(_bf(interp), _bf(w1b_ref[...]),
                    preferred_element_type=jnp.float32)
    h = jnp.maximum(h + b1_ref[...], zero)
    h = jnp.dot(_bf(h), _bf(w2_ref[...]), preferred_element_type=jnp.float32)
    h = jnp.maximum(h + b2_ref[...], zero)
    out_ref[0] = h


def kernel(xyz1, xyz2, points1, points2, W1, b1, W2, b2):
    B, N, _ = xyz1.shape
    S = xyz2.shape[1]
    D1 = points1.shape[2]
    D2 = points2.shape[2]
    F1 = W1.shape[1]
    F2 = W2.shape[1]
    nb = min(_NB, N)

    xyz2_t = jnp.transpose(xyz2, (0, 2, 1))
    w1a = W1[:D1]
    w1b = W1[D1:]
    b1r = b1.reshape(1, F1)
    b2r = b2.reshape(1, F2)

    out = pl.pallas_call(
        _fp_body,
        grid=(B, N // nb),
        in_specs=[
            pl.BlockSpec((1, nb, 3), lambda b, i: (b, i, 0)),
            pl.BlockSpec((1, 3, S), lambda b, i: (b, 0, 0)),
            pl.BlockSpec((1, nb, D1), lambda b, i: (b, i, 0)),
            pl.BlockSpec((1, S, D2), lambda b, i: (b, 0, 0)),
            pl.BlockSpec((D1, F1), lambda b, i: (0, 0)),
            pl.BlockSpec((D2, F1), lambda b, i: (0, 0)),
            pl.BlockSpec((1, F1), lambda b, i: (0, 0)),
            pl.BlockSpec((F1, F2), lambda b, i: (0, 0)),
            pl.BlockSpec((1, F2), lambda b, i: (0, 0)),
        ],
        out_specs=pl.BlockSpec((1, nb, F2), lambda b, i: (b, i, 0)),
        out_shape=jax.ShapeDtypeStruct((B, N, F2), jnp.float32),
    )(xyz1, xyz2_t, points1, points2, w1a, w1b, b1r, W2, b2r)
    return out


# speedup vs baseline: 31.8553x; 1.5518x over previous
import jax
import jax.numpy as jnp
from jax.experimental import pallas as pl

_NB = 512


def _fp_body(x1_ref, x2t_ref, p1_ref, p2h_ref, p2l_ref, w1a_ref, w1b_ref,
             b1_ref, w2_ref, b2_ref, out_ref):
    nb = x1_ref.shape[1]
    s = x2t_ref.shape[2]
    f32 = jnp.float32
    bf16 = jnp.bfloat16

    x1 = x1_ref[0]
    x2t = x2t_ref[0]
    a0 = x1[:, 0:1]
    a1 = x1[:, 1:2]
    a2 = x1[:, 2:3]
    c0 = x2t[0:1, :]
    c1 = x2t[1:2, :]
    c2 = x2t[2:3, :]

    dot = jnp.dot(x1.astype(bf16), x2t.astype(bf16),
                  preferred_element_type=f32)
    ss1 = a0 * a0 + a1 * a1 + a2 * a2
    ss2 = c0 * c0 + c1 * c1 + c2 * c2
    d = (-2.0 * dot + ss1) + ss2

    iota = jax.lax.broadcasted_iota(jnp.int32, (nb, s), 1).astype(f32)
    inf = f32(jnp.inf)
    sf = f32(s)

    m1 = jnp.min(d, axis=1, keepdims=True)
    i1 = jnp.min(jnp.where(d == m1, iota, sf), axis=1, keepdims=True)
    dm = jnp.where(iota == i1, inf, d)
    m2 = jnp.min(dm, axis=1, keepdims=True)
    i2 = jnp.min(jnp.where(dm == m2, iota, sf), axis=1, keepdims=True)
    dm = jnp.where(iota == i2, inf, dm)
    m3 = jnp.min(dm, axis=1, keepdims=True)
    i3 = jnp.min(jnp.where(dm == m3, iota, sf), axis=1, keepdims=True)

    r1 = 1.0 / (m1 + 1e-8)
    r2 = 1.0 / (m2 + 1e-8)
    r3 = 1.0 / (m3 + 1e-8)
    rn = r1 + r2 + r3
    w1 = r1 / rn
    w2 = r2 / rn
    w3 = r3 / rn

    zero = f32(0.0)
    wmat = (jnp.where(iota == i1, w1, zero)
            + jnp.where(iota == i2, w2, zero)
            + jnp.where(iota == i3, w3, zero))

    wm_hi = wmat.astype(bf16)
    wm_lo = (wmat - wm_hi.astype(f32)).astype(bf16)
    p2h = p2h_ref[0]
    interp = jnp.dot(wm_hi, p2h, preferred_element_type=f32)
    interp = interp + jnp.dot(wm_hi, p2l_ref[0], preferred_element_type=f32)
    interp = interp + jnp.dot(wm_lo, p2h, preferred_element_type=f32)

    h = jnp.dot(p1_ref[0].astype(bf16), w1a_ref[...],
                preferred_element_type=f32)
    h = h + jnp.dot(interp.astype(bf16), w1b_ref[...],
                    preferred_element_type=f32)
    h = jnp.maximum(h + b1_ref[...], zero)
    h = jnp.dot(h.astype(bf16), w2_ref[...], preferred_element_type=f32)
    h = jnp.maximum(h + b2_ref[...], zero)
    out_ref[0] = h


def kernel(xyz1, xyz2, points1, points2, W1, b1, W2, b2):
    B, N, _ = xyz1.shape
    S = xyz2.shape[1]
    D1 = points1.shape[2]
    D2 = points2.shape[2]
    F1 = W1.shape[1]
    F2 = W2.shape[1]
    nb = min(_NB, N)

    xyz2_t = jnp.transpose(xyz2, (0, 2, 1))
    p2_hi = points2.astype(jnp.bfloat16)
    p2_lo = (points2 - p2_hi.astype(jnp.float32)).astype(jnp.bfloat16)
    w1a = W1[:D1].astype(jnp.bfloat16)
    w1b = W1[D1:].astype(jnp.bfloat16)
    w2 = W2.astype(jnp.bfloat16)
    b1r = b1.reshape(1, F1)
    b2r = b2.reshape(1, F2)

    out = pl.pallas_call(
        _fp_body,
        grid=(B, N // nb),
        in_specs=[
            pl.BlockSpec((1, nb, 3), lambda b, i: (b, i, 0)),
            pl.BlockSpec((1, 3, S), lambda b, i: (b, 0, 0)),
            pl.BlockSpec((1, nb, D1), lambda b, i: (b, i, 0)),
            pl.BlockSpec((1, S, D2), lambda b, i: (b, 0, 0)),
            pl.BlockSpec((1, S, D2), lambda b, i: (b, 0, 0)),
            pl.BlockSpec((D1, F1), lambda b, i: (0, 0)),
            pl.BlockSpec((D2, F1), lambda b, i: (0, 0)),
            pl.BlockSpec((1, F1), lambda b, i: (0, 0)),
            pl.BlockSpec((F1, F2), lambda b, i: (0, 0)),
            pl.BlockSpec((1, F2), lambda b, i: (0, 0)),
        ],
        out_specs=pl.BlockSpec((1, nb, F2), lambda b, i: (b, i, 0)),
        out_shape=jax.ShapeDtypeStruct((B, N, F2), jnp.float32),
    )(xyz1, xyz2_t, points1, p2_hi, p2_lo, w1a, w1b, b1r, w2, b2r)
    return out


# speedup vs baseline: 36.8355x; 1.1563x over previous
import jax
import jax.numpy as jnp
from jax.experimental import pallas as pl

_NB = 512


def _fp_body(x1_ref, x2t_ref, p1_ref, p2h_ref, p2l_ref, w1a_ref, w1b_ref,
             b1_ref, w2_ref, b2_ref, out_ref):
    f32 = jnp.float32
    bf16 = jnp.bfloat16

    x1 = x1_ref[0]
    x2t = x2t_ref[0]
    a0 = x1[:, 0:1]
    a1 = x1[:, 1:2]
    a2 = x1[:, 2:3]
    c0 = x2t[0:1, :]
    c1 = x2t[1:2, :]
    c2 = x2t[2:3, :]

    dot2 = jnp.dot((-2.0 * x1).astype(bf16), x2t.astype(bf16),
                   preferred_element_type=f32)
    ss1 = a0 * a0 + a1 * a1 + a2 * a2
    ss2 = c0 * c0 + c1 * c1 + c2 * c2
    d = (dot2 + ss1) + ss2

    inf = f32(jnp.inf)
    m1 = jnp.min(d, axis=1, keepdims=True)
    msk1 = d == m1
    dm = jnp.where(msk1, inf, d)
    m2 = jnp.min(dm, axis=1, keepdims=True)
    msk2 = dm == m2
    dm = jnp.where(msk2, inf, dm)
    m3 = jnp.min(dm, axis=1, keepdims=True)
    msk3 = dm == m3

    r1 = 1.0 / (m1 + 1e-8)
    r2 = 1.0 / (m2 + 1e-8)
    r3 = 1.0 / (m3 + 1e-8)
    rn = r1 + r2 + r3
    w1 = r1 / rn
    w2 = r2 / rn
    w3 = r3 / rn

    zero = f32(0.0)
    wmat = (jnp.where(msk1, w1, zero) + jnp.where(msk2, w2, zero)
            + jnp.where(msk3, w3, zero))
    wm_hi = wmat.astype(bf16)
    wm_lo = (wmat - wm_hi.astype(f32)).astype(bf16)

    p2h = p2h_ref[0]
    interp = jnp.dot(wm_hi, p2h, preferred_element_type=f32)
    interp = interp + jnp.dot(wm_hi, p2l_ref[0], preferred_element_type=f32)
    interp = interp + jnp.dot(wm_lo, p2h, preferred_element_type=f32)

    h = jnp.dot(p1_ref[0].astype(bf16), w1a_ref[...],
                preferred_element_type=f32)
    h = h + jnp.dot(interp.astype(bf16), w1b_ref[...],
                    preferred_element_type=f32)
    h = jnp.maximum(h + b1_ref[...], zero)
    h = jnp.dot(h.astype(bf16), w2_ref[...], preferred_element_type=f32)
    h = jnp.maximum(h + b2_ref[...], zero)
    out_ref[0] = h


def kernel(xyz1, xyz2, points1, points2, W1, b1, W2, b2):
    B, N, _ = xyz1.shape
    S = xyz2.shape[1]
    D1 = points1.shape[2]
    D2 = points2.shape[2]
    F1 = W1.shape[1]
    F2 = W2.shape[1]
    nb = min(_NB, N)

    xyz2_t = jnp.transpose(xyz2, (0, 2, 1))
    p2_hi = points2.astype(jnp.bfloat16)
    p2_lo = (points2 - p2_hi.astype(jnp.float32)).astype(jnp.bfloat16)
    w1a = W1[:D1].astype(jnp.bfloat16)
    w1b = W1[D1:].astype(jnp.bfloat16)
    w2 = W2.astype(jnp.bfloat16)
    b1r = b1.reshape(1, F1)
    b2r = b2.reshape(1, F2)

    out = pl.pallas_call(
        _fp_body,
        grid=(B, N // nb),
        in_specs=[
            pl.BlockSpec((1, nb, 3), lambda b, i: (b, i, 0)),
            pl.BlockSpec((1, 3, S), lambda b, i: (b, 0, 0)),
            pl.BlockSpec((1, nb, D1), lambda b, i: (b, i, 0)),
            pl.BlockSpec((1, S, D2), lambda b, i: (b, 0, 0)),
            pl.BlockSpec((1, S, D2), lambda b, i: (b, 0, 0)),
            pl.BlockSpec((D1, F1), lambda b, i: (0, 0)),
            pl.BlockSpec((D2, F1), lambda b, i: (0, 0)),
            pl.BlockSpec((1, F1), lambda b, i: (0, 0)),
            pl.BlockSpec((F1, F2), lambda b, i: (0, 0)),
            pl.BlockSpec((1, F2), lambda b, i: (0, 0)),
        ],
        out_specs=pl.BlockSpec((1, nb, F2), lambda b, i: (b, i, 0)),
        out_shape=jax.ShapeDtypeStruct((B, N, F2), jnp.float32),
    )(xyz1, xyz2_t, points1, p2_hi, p2_lo, w1a, w1b, b1r, w2, b2r)
    return out


# speedup vs baseline: 40.3368x; 1.0951x over previous
import jax
import jax.numpy as jnp
from jax.experimental import pallas as pl

_NB = 512


def _fp_body(x1_ref, x2t_ref, p1_ref, p2h_ref, p2l_ref, w1a_ref, w1b_ref,
             b1_ref, w2_ref, b2_ref, out_ref):
    f32 = jnp.float32
    bf16 = jnp.bfloat16

    x1 = x1_ref[0]
    x2t = x2t_ref[0]
    a0 = x1[:, 0:1]
    a1 = x1[:, 1:2]
    a2 = x1[:, 2:3]
    c0 = x2t[0:1, :]
    c1 = x2t[1:2, :]
    c2 = x2t[2:3, :]

    dot2 = jnp.dot((-2.0 * x1).astype(bf16), x2t.astype(bf16),
                   preferred_element_type=f32)
    ss1 = a0 * a0 + a1 * a1 + a2 * a2
    ss2 = c0 * c0 + c1 * c1 + c2 * c2
    d = (dot2 + ss1) + ss2

    inf = f32(jnp.inf)
    m1 = jnp.min(d, axis=1, keepdims=True)
    msk1 = d == m1
    dm = jnp.where(msk1, inf, d)
    m2 = jnp.min(dm, axis=1, keepdims=True)
    msk2 = dm == m2
    dm = jnp.where(msk2, inf, dm)
    m3 = jnp.min(dm, axis=1, keepdims=True)
    msk3 = dm == m3

    r1 = 1.0 / (m1 + 1e-8)
    r2 = 1.0 / (m2 + 1e-8)
    r3 = 1.0 / (m3 + 1e-8)
    rn = r1 + r2 + r3
    w1 = r1 / rn
    w2 = r2 / rn
    w3 = r3 / rn

    zero = f32(0.0)
    wmat = (jnp.where(msk1, w1, zero) + jnp.where(msk2, w2, zero)
            + jnp.where(msk3, w3, zero))
    wm_hi = wmat.astype(bf16)

    p2h = p2h_ref[0]
    interp = jnp.dot(wm_hi, p2h, preferred_element_type=f32)
    interp = interp + jnp.dot(wm_hi, p2l_ref[0], preferred_element_type=f32)

    h = jnp.dot(p1_ref[0].astype(bf16), w1a_ref[...],
                preferred_element_type=f32)
    h = h + jnp.dot(interp.astype(bf16), w1b_ref[...],
                    preferred_element_type=f32)
    h = jnp.maximum(h + b1_ref[...], zero)
    h = jnp.dot(h.astype(bf16), w2_ref[...], preferred_element_type=f32)
    h = jnp.maximum(h + b2_ref[...], zero)
    out_ref[0] = h


def kernel(xyz1, xyz2, points1, points2, W1, b1, W2, b2):
    B, N, _ = xyz1.shape
    S = xyz2.shape[1]
    D1 = points1.shape[2]
    D2 = points2.shape[2]
    F1 = W1.shape[1]
    F2 = W2.shape[1]
    nb = min(_NB, N)

    xyz2_t = jnp.transpose(xyz2, (0, 2, 1))
    p2_hi = points2.astype(jnp.bfloat16)
    p2_lo = (points2 - p2_hi.astype(jnp.float32)).astype(jnp.bfloat16)
    w1a = W1[:D1].astype(jnp.bfloat16)
    w1b = W1[D1:].astype(jnp.bfloat16)
    w2 = W2.astype(jnp.bfloat16)
    b1r = b1.reshape(1, F1)
    b2r = b2.reshape(1, F2)

    out = pl.pallas_call(
        _fp_body,
        grid=(B, N // nb),
        in_specs=[
            pl.BlockSpec((1, nb, 3), lambda b, i: (b, i, 0)),
            pl.BlockSpec((1, 3, S), lambda b, i: (b, 0, 0)),
            pl.BlockSpec((1, nb, D1), lambda b, i: (b, i, 0)),
            pl.BlockSpec((1, S, D2), lambda b, i: (b, 0, 0)),
            pl.BlockSpec((1, S, D2), lambda b, i: (b, 0, 0)),
            pl.BlockSpec((D1, F1), lambda b, i: (0, 0)),
            pl.BlockSpec((D2, F1), lambda b, i: (0, 0)),
            pl.BlockSpec((1, F1), lambda b, i: (0, 0)),
            pl.BlockSpec((F1, F2), lambda b, i: (0, 0)),
            pl.BlockSpec((1, F2), lambda b, i: (0, 0)),
        ],
        out_specs=pl.BlockSpec((1, nb, F2), lambda b, i: (b, i, 0)),
        out_shape=jax.ShapeDtypeStruct((B, N, F2), jnp.float32),
    )(xyz1, xyz2_t, points1, p2_hi, p2_lo, w1a, w1b, b1r, w2, b2r)
    return out
